# baseline (speedup 1.0000x reference)
import functools

import jax
import jax.numpy as jnp
from jax import lax
from jax.experimental import pallas as pl
from jax.experimental.pallas import tpu as pltpu
from jax.experimental.pallas import tpu_sc as plsc

_TEMPERATURE = 1.0

_NC = 2
_NS = 16
_NW = _NC * _NS
_L = 16

_NT = 36
_NPAIRS = 4
_TS = 2 * _NPAIRS * _NT
_TC_BLOCK = 2048


def _sc_partials(logits2d, targets_flat, n_rows, v):
  rows_per_w = n_rows // _NW
  groups_per_w = rows_per_w // 8
  mesh = plsc.VectorSubcoreMesh(
      core_axis_name="c", subcore_axis_name="s",
      num_cores=_NC, num_subcores=_NS)

  @functools.partial(
      pl.kernel,
      out_type=(
          jax.ShapeDtypeStruct((n_rows,), jnp.float32),
          jax.ShapeDtypeStruct((n_rows,), jnp.float32),
      ),
      mesh=mesh,
      scratch_types=[
          pltpu.VMEM((8, _NT * 128), jnp.float32),
          pltpu.VMEM((8, _NT * 128), jnp.float32),
          pltpu.VMEM((rows_per_w,), jnp.int32),
          pltpu.VMEM((rows_per_w,), jnp.float32),
          pltpu.VMEM((rows_per_w,), jnp.float32),
          pltpu.SemaphoreType.DMA,
          pltpu.SemaphoreType.DMA,
      ],
      compiler_params=pltpu.CompilerParams(needs_layout_passes=False),
  )
  def partials_kernel(logits_hbm, targets_hbm, maxd_hbm, tval_hbm,
                      buf0, buf1, tvec, maxv, tvalv, sem0, sem1):
    wid = lax.axis_index("s") * _NC + lax.axis_index("c")
    row0 = wid * rows_per_w
    pltpu.sync_copy(targets_hbm.at[pl.ds(row0, rows_per_w)], tvec)
    lane = lax.iota(jnp.int32, _L)
    valid8 = lane < 8
    r_vec = lane & 7
    neg_inf = jnp.full((_L,), -jnp.inf, jnp.float32)

    def chunk_src(tr, c0):
      return logits_hbm.at[pl.ds(tr * 8, 8), pl.ds(c0 * 128, _NT * 128)]

    def process(buf, tv, c0, carry):
      accs, tval = carry
      in_chunk = (tv >= c0 * 128) & (tv < (c0 + _NT) * 128) & valid8
      col = jnp.where(in_chunk, tv - c0 * 128, 0)
      g = plsc.load_gather(buf, [r_vec, col])
      tval = jnp.maximum(tval, jnp.where(in_chunk, g, neg_inf))
      plsc.store_scatter(buf, [r_vec, col], neg_inf, mask=in_chunk)

      def body(t, a):
        new = []
        for r in range(8):
          ar = a[r]
          for u in range(2):
            for j in range(8):
              ar = jnp.maximum(
                  ar, buf[r, pl.ds((2 * t + u) * 128 + j * _L, _L)])
          new.append(ar)
        return tuple(new)

      accs = lax.fori_loop(0, _NT // 2, body, accs)
      return accs, tval

    def group_body(gi, carry):
      tr = wid * groups_per_w + gi
      tv = plsc.load_gather(tvec, [gi * 8 + r_vec])
      state = ((neg_inf,) * 8, neg_inf)
      pltpu.async_copy(chunk_src(tr, 0), buf0, sem0)

      def pair_body(k, st):
        c0a = (2 * k) * _NT
        c0b = (2 * k + 1) * _NT
        pltpu.async_copy(chunk_src(tr, c0b), buf1, sem1)
        pltpu.make_async_copy(chunk_src(tr, c0a), buf0, sem0).wait()
        st = process(buf0, tv, c0a, st)

        @pl.when(k < _NPAIRS - 1)
        def _():
          pltpu.async_copy(chunk_src(tr, c0b + _NT), buf0, sem0)

        pltpu.make_async_copy(chunk_src(tr, c0b), buf1, sem1).wait()
        return process(buf1, tv, c0b, st)

      accs, tval = lax.fori_loop(0, _NPAIRS, pair_body, state)

      maxd_vec = neg_inf
      for r in range(8):
        m = jnp.max(accs[r])
        maxd_vec = jnp.where(lane == r, jnp.full((_L,), m, jnp.float32),
                             maxd_vec)
      plsc.store_scatter(maxv, [gi * 8 + r_vec], maxd_vec, mask=valid8)
      plsc.store_scatter(tvalv, [gi * 8 + r_vec], tval, mask=valid8)
      return carry

    lax.fori_loop(0, groups_per_w, group_body, 0)
    pltpu.sync_copy(maxv, maxd_hbm.at[pl.ds(row0, rows_per_w)])
    pltpu.sync_copy(tvalv, tval_hbm.at[pl.ds(row0, rows_per_w)])

  return partials_kernel(logits2d, targets_flat)


def _tc_tail_partials(logits2d, targets, col0, v):
  n = logits2d.shape[0]
  w = v - col0
  assert col0 % _TC_BLOCK == 0
  nblocks = (w + _TC_BLOCK - 1) // _TC_BLOCK
  blk0 = col0 // _TC_BLOCK

  def body(x_ref, tgt_ref, max_ref, tval_ref):
    j = pl.program_id(0)

    @pl.when(j == 0)
    def _():
      max_ref[...] = jnp.full((n, 1), -jnp.inf, jnp.float32)
      tval_ref[...] = jnp.full((n, 1), -jnp.inf, jnp.float32)

    x = x_ref[...]
    cols = (col0 + j * _TC_BLOCK
            + lax.broadcasted_iota(jnp.int32, (n, _TC_BLOCK), 1))
    is_t = cols == tgt_ref[...]
    oob = cols >= v
    bmax = jnp.max(jnp.where(is_t | oob, -jnp.inf, x), axis=1,
                   keepdims=True)
    btval = jnp.max(jnp.where(is_t & ~oob, x, -jnp.inf), axis=1,
                    keepdims=True)
    max_ref[...] = jnp.maximum(max_ref[...], bmax)
    tval_ref[...] = jnp.maximum(tval_ref[...], btval)

  return pl.pallas_call(
      body,
      grid=(nblocks,),
      in_specs=[
          pl.BlockSpec((n, _TC_BLOCK), lambda j: (0, blk0 + j)),
          pl.BlockSpec((n, 1), lambda j: (0, 0)),
      ],
      out_specs=[
          pl.BlockSpec((n, 1), lambda j: (0, 0)),
          pl.BlockSpec((n, 1), lambda j: (0, 0)),
      ],
      out_shape=(
          jax.ShapeDtypeStruct((n, 1), jnp.float32),
          jax.ShapeDtypeStruct((n, 1), jnp.float32),
      ),
      compiler_params=pltpu.CompilerParams(
          dimension_semantics=("arbitrary",)),
  )(logits2d, targets.reshape(n, 1))


def _tc_combine(sc_max, sc_tval, tc_max, tc_tval):
  n = sc_max.shape[0]

  def body(a_ref, b_ref, c_ref, d_ref, o_ref):
    maxd = jnp.maximum(a_ref[...], c_ref[...])
    tval = jnp.maximum(b_ref[...], d_ref[...])
    margin = (tval - maxd) / _TEMPERATURE
    loss = -jnp.mean(jax.nn.log_sigmoid(margin))
    o_ref[...] = jnp.full((1, 1), loss, jnp.float32)

  out = pl.pallas_call(
      body,
      out_shape=jax.ShapeDtypeStruct((1, 1), jnp.float32),
  )(sc_max.reshape(n, 1), sc_tval.reshape(n, 1), tc_max, tc_tval)
  return out[0, 0]


@jax.jit
def kernel(logits, target_positions):
  b, t, v = logits.shape
  k = target_positions.shape[1]
  n = b * k
  logits2d = logits[:, :k, :].reshape(n, v)
  tflat = target_positions.reshape(-1).astype(jnp.int32)
  sc_max, sc_tval = _sc_partials(logits2d, tflat, n, v)
  tc_max, tc_tval = _tc_tail_partials(logits2d, tflat, 128 * _TS, v)
  return _tc_combine(sc_max, sc_tval, tc_max, tc_tval)


# speedup vs baseline: 1.0096x; 1.0096x over previous
import functools

import jax
import jax.numpy as jnp
from jax import lax
from jax.experimental import pallas as pl
from jax.experimental.pallas import tpu as pltpu
from jax.experimental.pallas import tpu_sc as plsc

_TEMPERATURE = 1.0

_NC = 2
_NS = 16
_NW = _NC * _NS
_L = 16

_NT = 40
_NPAIRS = 4
_TS = 2 * _NPAIRS * _NT
_TC_BLOCK = 2048


def _sc_partials(logits2d, targets_flat, n_rows, v):
  rows_per_w = n_rows // _NW
  groups_per_w = rows_per_w // 8
  mesh = plsc.VectorSubcoreMesh(
      core_axis_name="c", subcore_axis_name="s",
      num_cores=_NC, num_subcores=_NS)

  @functools.partial(
      pl.kernel,
      out_type=(
          jax.ShapeDtypeStruct((n_rows,), jnp.float32),
          jax.ShapeDtypeStruct((n_rows,), jnp.float32),
      ),
      mesh=mesh,
      scratch_types=[
          pltpu.VMEM((8, _NT * 128), jnp.float32),
          pltpu.VMEM((8, _NT * 128), jnp.float32),
          pltpu.VMEM((rows_per_w,), jnp.int32),
          pltpu.VMEM((rows_per_w,), jnp.float32),
          pltpu.VMEM((rows_per_w,), jnp.float32),
          pltpu.SemaphoreType.DMA,
          pltpu.SemaphoreType.DMA,
      ],
      compiler_params=pltpu.CompilerParams(needs_layout_passes=False),
  )
  def partials_kernel(logits_hbm, targets_hbm, maxd_hbm, tval_hbm,
                      buf0, buf1, tvec, maxv, tvalv, sem0, sem1):
    wid = lax.axis_index("s") * _NC + lax.axis_index("c")
    row0 = wid * rows_per_w
    pltpu.sync_copy(targets_hbm.at[pl.ds(row0, rows_per_w)], tvec)
    lane = lax.iota(jnp.int32, _L)
    valid8 = lane < 8
    r_vec = lane & 7
    neg_inf = jnp.full((_L,), -jnp.inf, jnp.float32)

    def chunk_src(tr, c0):
      return logits_hbm.at[pl.ds(tr * 8, 8), pl.ds(c0 * 128, _NT * 128)]

    def process(buf, tv, c0, carry):
      accs, tval = carry
      in_chunk = (tv >= c0 * 128) & (tv < (c0 + _NT) * 128) & valid8
      col = jnp.where(in_chunk, tv - c0 * 128, 0)
      g = plsc.load_gather(buf, [r_vec, col])
      tval = jnp.maximum(tval, jnp.where(in_chunk, g, neg_inf))
      plsc.store_scatter(buf, [r_vec, col], neg_inf, mask=in_chunk)

      def body(t, a):
        new = []
        for r in range(8):
          ar = a[r]
          for u in range(2):
            for j in range(8):
              ar = jnp.maximum(
                  ar, buf[r, pl.ds((2 * t + u) * 128 + j * _L, _L)])
          new.append(ar)
        return tuple(new)

      accs = lax.fori_loop(0, _NT // 2, body, accs)
      return accs, tval

    def group_body(gi, carry):
      tr = wid * groups_per_w + gi
      tv = plsc.load_gather(tvec, [gi * 8 + r_vec])
      state = ((neg_inf,) * 8, neg_inf)
      pltpu.async_copy(chunk_src(tr, 0), buf0, sem0)

      def pair_body(k, st):
        c0a = (2 * k) * _NT
        c0b = (2 * k + 1) * _NT
        pltpu.async_copy(chunk_src(tr, c0b), buf1, sem1)
        pltpu.make_async_copy(chunk_src(tr, c0a), buf0, sem0).wait()
        st = process(buf0, tv, c0a, st)

        @pl.when(k < _NPAIRS - 1)
        def _():
          pltpu.async_copy(chunk_src(tr, c0b + _NT), buf0, sem0)

        pltpu.make_async_copy(chunk_src(tr, c0b), buf1, sem1).wait()
        return process(buf1, tv, c0b, st)

      accs, tval = lax.fori_loop(0, _NPAIRS, pair_body, state)

      maxd_vec = neg_inf
      for r in range(8):
        m = jnp.max(accs[r])
        maxd_vec = jnp.where(lane == r, jnp.full((_L,), m, jnp.float32),
                             maxd_vec)
      plsc.store_scatter(maxv, [gi * 8 + r_vec], maxd_vec, mask=valid8)
      plsc.store_scatter(tvalv, [gi * 8 + r_vec], tval, mask=valid8)
      return carry

    lax.fori_loop(0, groups_per_w, group_body, 0)
    pltpu.sync_copy(maxv, maxd_hbm.at[pl.ds(row0, rows_per_w)])
    pltpu.sync_copy(tvalv, tval_hbm.at[pl.ds(row0, rows_per_w)])

  return partials_kernel(logits2d, targets_flat)


def _tc_tail_partials(logits2d, targets, col0, v):
  n = logits2d.shape[0]
  w = v - col0
  assert col0 % _TC_BLOCK == 0
  nblocks = (w + _TC_BLOCK - 1) // _TC_BLOCK
  blk0 = col0 // _TC_BLOCK

  def body(x_ref, tgt_ref, max_ref, tval_ref):
    j = pl.program_id(0)

    @pl.when(j == 0)
    def _():
      max_ref[...] = jnp.full((n, 1), -jnp.inf, jnp.float32)
      tval_ref[...] = jnp.full((n, 1), -jnp.inf, jnp.float32)

    x = x_ref[...]
    cols = (col0 + j * _TC_BLOCK
            + lax.broadcasted_iota(jnp.int32, (n, _TC_BLOCK), 1))
    is_t = cols == tgt_ref[...]
    oob = cols >= v
    bmax = jnp.max(jnp.where(is_t | oob, -jnp.inf, x), axis=1,
                   keepdims=True)
    btval = jnp.max(jnp.where(is_t & ~oob, x, -jnp.inf), axis=1,
                    keepdims=True)
    max_ref[...] = jnp.maximum(max_ref[...], bmax)
    tval_ref[...] = jnp.maximum(tval_ref[...], btval)

  return pl.pallas_call(
      body,
      grid=(nblocks,),
      in_specs=[
          pl.BlockSpec((n, _TC_BLOCK), lambda j: (0, blk0 + j)),
          pl.BlockSpec((n, 1), lambda j: (0, 0)),
      ],
      out_specs=[
          pl.BlockSpec((n, 1), lambda j: (0, 0)),
          pl.BlockSpec((n, 1), lambda j: (0, 0)),
      ],
      out_shape=(
          jax.ShapeDtypeStruct((n, 1), jnp.float32),
          jax.ShapeDtypeStruct((n, 1), jnp.float32),
      ),
      compiler_params=pltpu.CompilerParams(
          dimension_semantics=("arbitrary",)),
  )(logits2d, targets.reshape(n, 1))


def _tc_combine(sc_max, sc_tval, tc_max, tc_tval):
  n = sc_max.shape[0]

  def body(a_ref, b_ref, c_ref, d_ref, o_ref):
    maxd = jnp.maximum(a_ref[...], c_ref[...])
    tval = jnp.maximum(b_ref[...], d_ref[...])
    margin = (tval - maxd) / _TEMPERATURE
    loss = -jnp.mean(jax.nn.log_sigmoid(margin))
    o_ref[...] = jnp.full((1, 1), loss, jnp.float32)

  out = pl.pallas_call(
      body,
      out_shape=jax.ShapeDtypeStruct((1, 1), jnp.float32),
  )(sc_max.reshape(n, 1), sc_tval.reshape(n, 1), tc_max, tc_tval)
  return out[0, 0]


@jax.jit
def kernel(logits, target_positions):
  b, t, v = logits.shape
  k = target_positions.shape[1]
  n = b * k
  logits2d = logits[:, :k, :].reshape(n, v)
  tflat = target_positions.reshape(-1).astype(jnp.int32)
  sc_max, sc_tval = _sc_partials(logits2d, tflat, n, v)
  tc_max, tc_tval = _tc_tail_partials(logits2d, tflat, 128 * _TS, v)
  return _tc_combine(sc_max, sc_tval, tc_max, tc_tval)


# speedup vs baseline: 1.0448x; 1.0349x over previous
import functools

import jax
import jax.numpy as jnp
from jax import lax
from jax.experimental import pallas as pl
from jax.experimental.pallas import tpu as pltpu
from jax.experimental.pallas import tpu_sc as plsc

_TEMPERATURE = 1.0

_NC = 2
_NS = 16
_NW = _NC * _NS
_L = 16

_NT = 42
_NPAIRS = 4
_TS = 2 * _NPAIRS * _NT
_TC_BLOCK = 2048


def _sc_partials(logits2d, targets_flat, n_rows, v):
  rows_per_w = n_rows // _NW
  groups_per_w = rows_per_w // 8
  mesh = plsc.VectorSubcoreMesh(
      core_axis_name="c", subcore_axis_name="s",
      num_cores=_NC, num_subcores=_NS)

  @functools.partial(
      pl.kernel,
      out_type=(
          jax.ShapeDtypeStruct((n_rows,), jnp.float32),
          jax.ShapeDtypeStruct((n_rows,), jnp.float32),
      ),
      mesh=mesh,
      scratch_types=[
          pltpu.VMEM((8, _NT * 128), jnp.float32),
          pltpu.VMEM((8, _NT * 128), jnp.float32),
          pltpu.VMEM((rows_per_w,), jnp.int32),
          pltpu.VMEM((rows_per_w,), jnp.float32),
          pltpu.VMEM((rows_per_w,), jnp.float32),
          pltpu.SemaphoreType.DMA,
          pltpu.SemaphoreType.DMA,
      ],
      compiler_params=pltpu.CompilerParams(needs_layout_passes=False),
  )
  def partials_kernel(logits_hbm, targets_hbm, maxd_hbm, tval_hbm,
                      buf0, buf1, tvec, maxv, tvalv, sem0, sem1):
    wid = lax.axis_index("s") * _NC + lax.axis_index("c")
    row0 = wid * rows_per_w
    pltpu.sync_copy(targets_hbm.at[pl.ds(row0, rows_per_w)], tvec)
    lane = lax.iota(jnp.int32, _L)
    valid8 = lane < 8
    r_vec = lane & 7
    neg_inf = jnp.full((_L,), -jnp.inf, jnp.float32)

    def chunk_src(tr, c0):
      return logits_hbm.at[pl.ds(tr * 8, 8), pl.ds(c0 * 128, _NT * 128)]

    def process(buf, tv, c0, carry):
      accs, tval = carry
      in_chunk = (tv >= c0 * 128) & (tv < (c0 + _NT) * 128) & valid8
      col = jnp.where(in_chunk, tv - c0 * 128, 0)
      g = plsc.load_gather(buf, [r_vec, col])
      tval = jnp.maximum(tval, jnp.where(in_chunk, g, neg_inf))
      plsc.store_scatter(buf, [r_vec, col], neg_inf, mask=in_chunk)

      def body(t, a):
        new = []
        for r in range(8):
          ar = a[r]
          for u in range(2):
            for j in range(8):
              ar = jnp.maximum(
                  ar, buf[r, pl.ds((2 * t + u) * 128 + j * _L, _L)])
          new.append(ar)
        return tuple(new)

      accs = lax.fori_loop(0, _NT // 2, body, accs)
      return accs, tval

    def group_body(gi, carry):
      tr = wid * groups_per_w + gi
      tv = plsc.load_gather(tvec, [gi * 8 + r_vec])
      state = ((neg_inf,) * 8, neg_inf)
      pltpu.async_copy(chunk_src(tr, 0), buf0, sem0)

      def pair_body(k, st):
        c0a = (2 * k) * _NT
        c0b = (2 * k + 1) * _NT
        pltpu.async_copy(chunk_src(tr, c0b), buf1, sem1)
        pltpu.make_async_copy(chunk_src(tr, c0a), buf0, sem0).wait()
        st = process(buf0, tv, c0a, st)

        @pl.when(k < _NPAIRS - 1)
        def _():
          pltpu.async_copy(chunk_src(tr, c0b + _NT), buf0, sem0)

        pltpu.make_async_copy(chunk_src(tr, c0b), buf1, sem1).wait()
        return process(buf1, tv, c0b, st)

      accs, tval = lax.fori_loop(0, _NPAIRS, pair_body, state)

      maxd_vec = neg_inf
      for r in range(8):
        m = jnp.max(accs[r])
        maxd_vec = jnp.where(lane == r, jnp.full((_L,), m, jnp.float32),
                             maxd_vec)
      plsc.store_scatter(maxv, [gi * 8 + r_vec], maxd_vec, mask=valid8)
      plsc.store_scatter(tvalv, [gi * 8 + r_vec], tval, mask=valid8)
      return carry

    lax.fori_loop(0, groups_per_w, group_body, 0)
    pltpu.sync_copy(maxv, maxd_hbm.at[pl.ds(row0, rows_per_w)])
    pltpu.sync_copy(tvalv, tval_hbm.at[pl.ds(row0, rows_per_w)])

  return partials_kernel(logits2d, targets_flat)


def _tc_tail_partials(logits2d, targets, col0, v):
  n = logits2d.shape[0]
  w = v - col0
  assert col0 % _TC_BLOCK == 0
  nblocks = (w + _TC_BLOCK - 1) // _TC_BLOCK
  blk0 = col0 // _TC_BLOCK

  def body(x_ref, tgt_ref, max_ref, tval_ref, amax, atval):
    j = pl.program_id(0)

    @pl.when(j == 0)
    def _():
      amax[...] = jnp.full((n, 1), -jnp.inf, jnp.float32)
      atval[...] = jnp.full((n, 1), -jnp.inf, jnp.float32)

    x = x_ref[...]
    cols = (col0 + j * _TC_BLOCK
            + lax.broadcasted_iota(jnp.int32, (n, _TC_BLOCK), 1))
    is_t = cols == tgt_ref[...]
    oob = cols >= v
    bmax = jnp.max(jnp.where(is_t | oob, -jnp.inf, x), axis=1,
                   keepdims=True)
    btval = jnp.max(jnp.where(is_t & ~oob, x, -jnp.inf), axis=1,
                    keepdims=True)
    amax[...] = jnp.maximum(amax[...], bmax)
    atval[...] = jnp.maximum(atval[...], btval)

    @pl.when(j == nblocks - 1)
    def _():
      max_ref[...] = jnp.reshape(amax[...], (n // 128, 128))
      tval_ref[...] = jnp.reshape(atval[...], (n // 128, 128))

  return pl.pallas_call(
      body,
      grid=(nblocks,),
      in_specs=[
          pl.BlockSpec((n, _TC_BLOCK), lambda j: (0, blk0 + j)),
          pl.BlockSpec((n, 1), lambda j: (0, 0)),
      ],
      out_specs=[
          pl.BlockSpec((n // 128, 128), lambda j: (0, 0)),
          pl.BlockSpec((n // 128, 128), lambda j: (0, 0)),
      ],
      out_shape=(
          jax.ShapeDtypeStruct((n // 128, 128), jnp.float32),
          jax.ShapeDtypeStruct((n // 128, 128), jnp.float32),
      ),
      scratch_shapes=[
          pltpu.VMEM((n, 1), jnp.float32),
          pltpu.VMEM((n, 1), jnp.float32),
      ],
      compiler_params=pltpu.CompilerParams(
          dimension_semantics=("arbitrary",)),
  )(logits2d, targets.reshape(n, 1))


def _tc_combine(sc_max, sc_tval, tc_max, tc_tval):
  n = sc_max.shape[0]

  def body(a_ref, b_ref, c_ref, d_ref, o_ref):
    maxd = jnp.maximum(a_ref[...], c_ref[...])
    tval = jnp.maximum(b_ref[...], d_ref[...])
    margin = (tval - maxd) / _TEMPERATURE
    loss = -jnp.mean(jax.nn.log_sigmoid(margin))
    o_ref[...] = jnp.full((1, 1), loss, jnp.float32)

  out = pl.pallas_call(
      body,
      out_shape=jax.ShapeDtypeStruct((1, 1), jnp.float32),
  )(sc_max.reshape(n // 128, 128), sc_tval.reshape(n // 128, 128),
    tc_max, tc_tval)
  return out[0, 0]


@jax.jit
def kernel(logits, target_positions):
  b, t, v = logits.shape
  k = target_positions.shape[1]
  n = b * k
  logits2d = logits[:, :k, :].reshape(n, v)
  tflat = target_positions.reshape(-1).astype(jnp.int32)
  sc_max, sc_tval = _sc_partials(logits2d, tflat, n, v)
  tc_max, tc_tval = _tc_tail_partials(logits2d, tflat, 128 * _TS, v)
  return _tc_combine(sc_max, sc_tval, tc_max, tc_tval)
